# baseline (speedup 1.0000x reference)
import functools

import jax
import jax.numpy as jnp
from jax import lax
from jax.experimental import pallas as pl
from jax.experimental.pallas import tpu as pltpu
from jax.experimental.pallas import tpu_sc as plsc

_NC = 2
_NS = 16
_NW = _NC * _NS
_CHUNK = 128
_LANES = 16
_TBLK = 4096


def _transpose_kernel(tt_ref, out_ref):
    out_ref[...] = jnp.transpose(tt_ref[...])


@functools.partial(jax.jit, static_argnames=("batch", "dim"))
def _gather_add(h2, r2, table_t, *, batch, dim):
    num = table_t.shape[1]
    rowmajor = pl.pallas_call(
        _transpose_kernel,
        grid=((num + _TBLK - 1) // _TBLK,),
        in_specs=[pl.BlockSpec((dim, _TBLK), lambda j: (0, j))],
        out_specs=pl.BlockSpec((_TBLK, dim), lambda j: (j, 0)),
        out_shape=jax.ShapeDtypeStruct((num, dim), jnp.float32),
        compiler_params=pltpu.CompilerParams(
            dimension_semantics=("parallel",)),
    )(table_t)

    bpw = batch // _NW
    n_chunks = bpw // _CHUNK
    hp = bpw // 2
    vecs = dim // _LANES

    mesh = plsc.VectorSubcoreMesh(core_axis_name="c", subcore_axis_name="s")

    @functools.partial(
        pl.kernel,
        mesh=mesh,
        out_type=jax.ShapeDtypeStruct((batch, dim), jnp.float32),
        scratch_types=[
            pltpu.VMEM((n_chunks, _CHUNK), jnp.int32),
            pltpu.VMEM((n_chunks, _CHUNK), jnp.int32),
            pltpu.VMEM((hp, dim), jnp.float32),
            pltpu.VMEM((hp, dim), jnp.float32),
            pltpu.SemaphoreType.DMA,
            pltpu.SemaphoreType.DMA,
        ],
    )
    def k(h_hbm, r_hbm, tbl_hbm, out_hbm, hidx_v, ridx_v, rows_h, rows_r,
          semh, semr):
        wid = lax.axis_index("s") * _NC + lax.axis_index("c")
        pltpu.sync_copy(h_hbm.at[pl.ds(wid * n_chunks, n_chunks)], hidx_v)
        pltpu.sync_copy(r_hbm.at[pl.ds(wid * n_chunks, n_chunks)], ridx_v)

        groups_per_chunk = _CHUNK // _LANES

        for half in range(2):
            gbase = half * (hp // _LANES)

            def fire(g, _):
                gg = gbase + g
                c = gg // groups_per_chunk
                o = (gg % groups_per_chunk) * _LANES
                hvec = hidx_v[c, pl.ds(o, _LANES)]
                rvec = ridx_v[c, pl.ds(o, _LANES)]
                base = g * _LANES
                for j in range(_LANES):
                    pltpu.async_copy(tbl_hbm.at[pl.ds(hvec[j], 1)],
                                     rows_h.at[pl.ds(base + j, 1)], semh)
                    pltpu.async_copy(tbl_hbm.at[pl.ds(rvec[j], 1)],
                                     rows_r.at[pl.ds(base + j, 1)], semr)
                return 0

            lax.fori_loop(0, hp // _LANES, fire, 0)
            pltpu.make_async_copy(tbl_hbm.at[pl.ds(0, hp)], rows_h,
                                  semh).wait()
            pltpu.make_async_copy(tbl_hbm.at[pl.ds(0, hp)], rows_r,
                                  semr).wait()

            def body(i, _):
                for v in range(vecs):
                    sl = pl.ds(v * _LANES, _LANES)
                    rows_h[i, sl] = rows_h[i, sl] + rows_r[i, sl]
                return 0

            lax.fori_loop(0, hp, body, 0)
            pltpu.sync_copy(rows_h,
                            out_hbm.at[pl.ds(wid * bpw + half * hp, hp)])

    return k(h2, r2, rowmajor)


def kernel(h_idx, r_idx, t_idx, table):
    del t_idx
    batch = h_idx.shape[0]
    dim = table.shape[1]
    h2 = h_idx.astype(jnp.int32).reshape(-1, _CHUNK)
    r2 = r_idx.astype(jnp.int32).reshape(-1, _CHUNK)
    return _gather_add(h2, r2, table.T, batch=batch, dim=dim)


# speedup vs baseline: 1.0055x; 1.0055x over previous
import functools

import jax
import jax.numpy as jnp
from jax import lax
from jax.experimental import pallas as pl
from jax.experimental.pallas import tpu as pltpu
from jax.experimental.pallas import tpu_sc as plsc

_NC = 2
_NS = 16
_NW = _NC * _NS
_CHUNK = 128
_LANES = 16
_TBLK = 4096


def _transpose_kernel(tt_ref, out_ref):
    t = jnp.transpose(tt_ref[...])
    dim = t.shape[1]
    half = _TBLK // 2
    out_ref[:, 0:dim] = t[0:half, :]
    out_ref[:, dim:2 * dim] = t[half:_TBLK, :]


@functools.partial(jax.jit, static_argnames=("batch", "dim"))
def _gather_add(h2, r2, ho2, ro2, table_t, *, batch, dim):
    num = table_t.shape[1]
    nblk = (num + _TBLK - 1) // _TBLK
    half = _TBLK // 2
    packed = pl.pallas_call(
        _transpose_kernel,
        grid=(nblk,),
        in_specs=[pl.BlockSpec((dim, _TBLK), lambda j: (0, j))],
        out_specs=pl.BlockSpec((half, 2 * dim), lambda j: (j, 0)),
        out_shape=jax.ShapeDtypeStruct((nblk * half, 2 * dim), jnp.float32),
        compiler_params=pltpu.CompilerParams(
            dimension_semantics=("parallel",)),
    )(table_t)

    bpw = batch // _NW
    n_chunks = bpw // _CHUNK
    hp = bpw // 2
    vecs = dim // _LANES

    mesh = plsc.VectorSubcoreMesh(core_axis_name="c", subcore_axis_name="s")

    @functools.partial(
        pl.kernel,
        mesh=mesh,
        out_type=jax.ShapeDtypeStruct((batch, dim), jnp.float32),
        compiler_params=pltpu.CompilerParams(needs_layout_passes=False),
        scratch_types=[
            pltpu.VMEM((n_chunks, _CHUNK), jnp.int32),
            pltpu.VMEM((n_chunks, _CHUNK), jnp.int32),
            pltpu.VMEM((n_chunks, _CHUNK), jnp.int32),
            pltpu.VMEM((n_chunks, _CHUNK), jnp.int32),
            pltpu.VMEM((hp, 2 * dim), jnp.float32),
            pltpu.VMEM((hp, 2 * dim), jnp.float32),
            pltpu.VMEM((hp, dim), jnp.float32),
            pltpu.SemaphoreType.DMA,
        ],
    )
    def k(h_hbm, r_hbm, ho_hbm, ro_hbm, tp_hbm, out_hbm, hidx_v, ridx_v,
          hoff_v, roff_v, buf_h, buf_r, acc, sem):
        wid = lax.axis_index("s") * _NC + lax.axis_index("c")
        sl_w = pl.ds(wid * n_chunks, n_chunks)
        pltpu.sync_copy(h_hbm.at[sl_w], hidx_v)
        pltpu.sync_copy(r_hbm.at[sl_w], ridx_v)
        pltpu.sync_copy(ho_hbm.at[sl_w], hoff_v)
        pltpu.sync_copy(ro_hbm.at[sl_w], roff_v)

        lane_iota = lax.iota(jnp.int32, _LANES)

        for phase in range(2):
            cbase = phase * (hp // _CHUNK)
            copies = []
            for j in range(hp // _CHUNK):
                dst = pl.ds(j * _CHUNK, _CHUNK)
                copies.append(pltpu.async_copy(
                    tp_hbm.at[hidx_v.at[cbase + j]], buf_h.at[dst], sem))
                copies.append(pltpu.async_copy(
                    tp_hbm.at[ridx_v.at[cbase + j]], buf_r.at[dst], sem))
            for cp in copies:
                cp.wait()

            def body(g, _):
                c = (phase * hp + g * _LANES) // _CHUNK
                o = (phase * hp + g * _LANES) % _CHUNK
                hoff = hoff_v[c, pl.ds(o, _LANES)]
                roff = roff_v[c, pl.ds(o, _LANES)]
                base = g * _LANES
                for j in range(_LANES):
                    row = jnp.full((_LANES,), base + j, jnp.int32)
                    for v in range(vecs):
                        hl = plsc.load_gather(
                            buf_h, [row, hoff[j] + v * _LANES + lane_iota])
                        rl = plsc.load_gather(
                            buf_r, [row, roff[j] + v * _LANES + lane_iota])
                        acc[base + j, pl.ds(v * _LANES, _LANES)] = hl + rl
                return 0

            lax.fori_loop(0, hp // _LANES, body, 0)
            pltpu.sync_copy(acc,
                            out_hbm.at[pl.ds(wid * bpw + phase * hp, hp)])

    return k(h2, r2, ho2, ro2, packed)


def kernel(h_idx, r_idx, t_idx, table):
    del t_idx
    batch = h_idx.shape[0]
    dim = table.shape[1]
    half = _TBLK // 2

    def split(idx):
        i = idx.astype(jnp.int32)
        row = (i // _TBLK) * half + (i % half)
        off = ((i // half) % 2) * dim
        return (row.reshape(-1, _CHUNK), off.reshape(-1, _CHUNK))

    h2, ho2 = split(h_idx)
    r2, ro2 = split(r_idx)
    return _gather_add(h2, r2, ho2, ro2, table.T, batch=batch, dim=dim)
